# baseline (speedup 1.0000x reference)
import functools

import jax
import jax.numpy as jnp
from jax import lax
from jax.experimental import pallas as pl
from jax.experimental.pallas import tpu as pltpu
from jax.experimental.pallas import tpu_sc as plsc

D_MODEL = 64
SCALE = 8.0

NC = 2
NS = 16
NW = NC * NS

B = 4096 * 200
IDX_MINOR = 128
SUBS_PER_CHUNK = 8
CHUNK = IDX_MINOR * SUBS_PER_CHUNK
PER_W = B // NW
CHUNKS = PER_W // CHUNK
XROWS_PER_W = PER_W // IDX_MINOR


def _body(x_hbm, table_hbm, out_hbm, idx_v, rows_v, sem):
    wid = lax.axis_index("s") * NC + lax.axis_index("c")

    def chunk(c, carry):
        xrow0 = wid * XROWS_PER_W + c * SUBS_PER_CHUNK
        pltpu.sync_copy(x_hbm.at[pl.ds(xrow0, SUBS_PER_CHUNK)], idx_v)
        copies = [
            pltpu.async_copy(
                table_hbm.at[idx_v.at[j]],
                rows_v.at[pl.ds(j * IDX_MINOR, IDX_MINOR)],
                sem,
            )
            for j in range(SUBS_PER_CHUNK)
        ]
        for cp in copies:
            cp.wait()

        def mulrow(r, carry2):
            r0 = r * 2
            for rr in range(2):
                for s in range(D_MODEL // 16):
                    v = rows_v[r0 + rr, pl.ds(s * 16, 16)]
                    rows_v[r0 + rr, pl.ds(s * 16, 16)] = v * SCALE
            return carry2

        lax.fori_loop(0, CHUNK // 2, mulrow, 0)

        out0 = wid * PER_W + c * CHUNK
        pltpu.sync_copy(rows_v, out_hbm.at[pl.ds(out0, CHUNK)])
        return carry

    lax.fori_loop(0, CHUNKS, chunk, 0)


@functools.partial(jax.jit, static_argnames=())
def _run(x2d, table):
    mesh = plsc.VectorSubcoreMesh(core_axis_name="c", subcore_axis_name="s")
    f = pl.kernel(
        _body,
        mesh=mesh,
        out_type=jax.ShapeDtypeStruct((B, D_MODEL), jnp.float32),
        scratch_types=[
            pltpu.VMEM((SUBS_PER_CHUNK, IDX_MINOR), jnp.int32),
            pltpu.VMEM((CHUNK, D_MODEL), jnp.float32),
            pltpu.SemaphoreType.DMA,
        ],
        compiler_params=pltpu.CompilerParams(use_tc_tiling_on_sc=False),
    )
    return f(x2d, table)


def kernel(x, table):
    x2d = x.reshape(B // IDX_MINOR, IDX_MINOR).astype(jnp.int32)
    out = _run(x2d, table)
    return out.reshape(x.shape[0], x.shape[1], D_MODEL)


# speedup vs baseline: 1.0605x; 1.0605x over previous
import functools

import jax
import jax.numpy as jnp
from jax import lax
from jax.experimental import pallas as pl
from jax.experimental.pallas import tpu as pltpu
from jax.experimental.pallas import tpu_sc as plsc

D_MODEL = 64
SCALE = 8.0

NC = 2
NS = 16
NW = NC * NS

B = 4096 * 200
IDX_MINOR = 128
CHUNK = 256
SUBS = CHUNK // IDX_MINOR
NBUF = 4
PER_W = B // NW
NCH = PER_W // CHUNK
OUTER = NCH // NBUF
XROWS_PER_W = PER_W // IDX_MINOR


def _body(x_hbm, table_hbm, out_hbm, idx_all, rows, gsem, dsem):
    wid = lax.axis_index("s") * NC + lax.axis_index("c")

    pltpu.sync_copy(x_hbm.at[pl.ds(wid * XROWS_PER_W, XROWS_PER_W)], idx_all)

    def fire_gather(k, b):
        for j in range(SUBS):
            pltpu.async_copy(
                table_hbm.at[idx_all.at[k * SUBS + j]],
                rows.at[b, pl.ds(j * IDX_MINOR, IDX_MINOR)],
                gsem.at[b],
            )

    def wait_gather(b):
        pltpu.make_async_copy(
            table_hbm.at[pl.ds(0, CHUNK)], rows.at[b], gsem.at[b]
        ).wait()

    def fire_drain(k, b):
        pltpu.async_copy(
            rows.at[b],
            out_hbm.at[pl.ds((wid * NCH + k) * CHUNK, CHUNK)],
            dsem.at[b],
        )

    def wait_drain(b):
        pltpu.make_async_copy(
            rows.at[b], out_hbm.at[pl.ds(0, CHUNK)], dsem.at[b]
        ).wait()

    for b in range(NBUF - 1):
        fire_gather(b, b)

    def outer(g, carry):
        for b in range(NBUF):
            k = g * NBUF + b
            bn = (b + NBUF - 1) % NBUF

            @pl.when(k >= 1)
            def _():
                wait_drain(bn)

            @pl.when(k + NBUF - 1 < NCH)
            def _():
                fire_gather(k + NBUF - 1, bn)

            wait_gather(b)

            @plsc.parallel_loop(0, CHUNK, step=2, unroll=4)
            def _mul(r):
                for rr in range(2):
                    for s in range(D_MODEL // 16):
                        v = rows[b, r + rr, pl.ds(s * 16, 16)]
                        rows[b, r + rr, pl.ds(s * 16, 16)] = v * SCALE

            fire_drain(k, b)
        return carry

    lax.fori_loop(0, OUTER, outer, 0)
    wait_drain((NCH - 1) % NBUF)


@jax.jit
def _run(x2d, table):
    mesh = plsc.VectorSubcoreMesh(core_axis_name="c", subcore_axis_name="s")
    f = pl.kernel(
        _body,
        mesh=mesh,
        out_type=jax.ShapeDtypeStruct((B, D_MODEL), jnp.float32),
        scratch_types=[
            pltpu.VMEM((XROWS_PER_W, IDX_MINOR), jnp.int32),
            pltpu.VMEM((NBUF, CHUNK, D_MODEL), jnp.float32),
            pltpu.SemaphoreType.DMA((NBUF,)),
            pltpu.SemaphoreType.DMA((NBUF,)),
        ],
        compiler_params=pltpu.CompilerParams(use_tc_tiling_on_sc=False),
    )
    return f(x2d, table)


def kernel(x, table):
    x2d = x.reshape(B // IDX_MINOR, IDX_MINOR).astype(jnp.int32)
    out = _run(x2d, table)
    return out.reshape(x.shape[0], x.shape[1], D_MODEL)
